# baseline (speedup 1.0000x reference)
import functools

import jax
import jax.numpy as jnp
from jax import lax
from jax.experimental import pallas as pl
from jax.experimental.pallas import tpu as pltpu
from jax.experimental.pallas import tpu_sc as plsc

NC = 2
NS = 16
NW = NC * NS
CHUNK = 128


def _sc_aggregate(x, ei4, zeros, n_pad):
    _, d = x.shape
    cpw = ei4.shape[1] - 2
    rps = n_pad // NS

    mesh = plsc.VectorSubcoreMesh(core_axis_name="c", subcore_axis_name="s")

    @functools.partial(
        pl.kernel,
        out_type=jax.ShapeDtypeStruct((NC, n_pad, d), jnp.float32),
        mesh=mesh,
        scratch_types=[
            pltpu.VMEM((2, CHUNK), jnp.int32),
            pltpu.VMEM((2, CHUNK), jnp.int32),
            pltpu.VMEM((CHUNK, d), jnp.float32),
            pltpu.VMEM((CHUNK, d), jnp.float32),
            pltpu.VMEM_SHARED((n_pad, d), jnp.float32),
            pltpu.SemaphoreType.DMA,
            pltpu.SemaphoreType.DMA,
        ],
    )
    def sc_agg(x_hbm, ei_hbm, z_hbm, out_hbm,
               ib0, ib1, rows0, rows1, acc, sem0, sem1):
        c = lax.axis_index("c")
        s = lax.axis_index("s")
        wid = c * NS + s
        pltpu.sync_copy(z_hbm, acc.at[pl.ds(s * rps, rps)])
        plsc.subcore_barrier()

        pltpu.sync_copy(ei_hbm.at[wid, 0], ib0)
        pltpu.async_copy(x_hbm.at[ib0.at[0]], rows0, sem0)
        pltpu.sync_copy(ei_hbm.at[wid, 1], ib1)
        pltpu.async_copy(x_hbm.at[ib1.at[0]], rows1, sem1)

        def body(jj, carry):
            ja = 2 * jj
            pltpu.make_async_copy(x_hbm.at[ib0.at[0]], rows0, sem0).wait()
            pltpu.sync_copy(rows0, acc.at[ib0.at[1]], add=True)
            pltpu.sync_copy(ei_hbm.at[wid, ja + 2], ib0)
            pltpu.async_copy(x_hbm.at[ib0.at[0]], rows0, sem0)
            pltpu.make_async_copy(x_hbm.at[ib1.at[0]], rows1, sem1).wait()
            pltpu.sync_copy(rows1, acc.at[ib1.at[1]], add=True)
            pltpu.sync_copy(ei_hbm.at[wid, ja + 3], ib1)
            pltpu.async_copy(x_hbm.at[ib1.at[0]], rows1, sem1)
            return carry

        lax.fori_loop(0, cpw // 2, body, 0)
        pltpu.make_async_copy(x_hbm.at[ib0.at[0]], rows0, sem0).wait()
        pltpu.make_async_copy(x_hbm.at[ib1.at[0]], rows1, sem1).wait()

        plsc.subcore_barrier()
        pltpu.sync_copy(acc.at[pl.ds(s * rps, rps)],
                        out_hbm.at[c].at[pl.ds(s * rps, rps)])

    return sc_agg(x, ei4, zeros)


def _tc_mlp(x, parts, W1, b1, W2, b2, alpha):
    n, d = x.shape
    do = W2.shape[1]
    br = 1000

    def body(x_ref, p_ref, w1_ref, b1_ref, w2_ref, b2_ref, a_ref, o_ref):
        h = x_ref[...] + p_ref[0] + p_ref[1]
        h = jnp.dot(h, w1_ref[...], preferred_element_type=jnp.float32) + b1_ref[...]
        h = jnp.maximum(h, 0.0)
        y = jnp.dot(h, w2_ref[...], preferred_element_type=jnp.float32) + b2_ref[...]
        o_ref[...] = y * a_ref[0, 0]

    return pl.pallas_call(
        body,
        grid=(n // br,),
        in_specs=[
            pl.BlockSpec((br, d), lambda i: (i, 0)),
            pl.BlockSpec((NC, br, d), lambda i: (0, i, 0)),
            pl.BlockSpec((d, do), lambda i: (0, 0)),
            pl.BlockSpec((1, do), lambda i: (0, 0)),
            pl.BlockSpec((do, do), lambda i: (0, 0)),
            pl.BlockSpec((1, do), lambda i: (0, 0)),
            pl.BlockSpec((1, 1), lambda i: (0, 0)),
        ],
        out_specs=pl.BlockSpec((br, do), lambda i: (i, 0)),
        out_shape=jax.ShapeDtypeStruct((n, do), jnp.float32),
    )(x, parts, W1, b1.reshape(1, do), W2, b2.reshape(1, do), alpha.reshape(1, 1))


def kernel(x, edge_index, W1, b1, W2, b2, alpha):
    n, d = x.shape
    e = edge_index.shape[1]

    cpw = -(-e // (NW * CHUNK))
    cpw += cpw % 2
    e_pad = NW * cpw * CHUNK
    n_pad = -(-(n + 1) // (NS * 8)) * (NS * 8)

    src = edge_index[0].astype(jnp.int32)
    dst = edge_index[1].astype(jnp.int32)
    pad = e_pad - e
    src = jnp.concatenate([src, jnp.zeros((pad,), jnp.int32)])
    pad_dst = n + jnp.arange(pad, dtype=jnp.int32) % (n_pad - n)
    dst = jnp.concatenate([dst, pad_dst])
    ei4 = jnp.stack([src.reshape(NW, cpw, CHUNK), dst.reshape(NW, cpw, CHUNK)],
                    axis=2)
    ei4 = jnp.pad(ei4, ((0, 0), (0, 2), (0, 0), (0, 0)))
    zeros = jnp.zeros((n_pad // NS, d), jnp.float32)

    parts = _sc_aggregate(x, ei4, zeros, n_pad)
    y = _tc_mlp(x, parts, W1, b1, W2, b2, alpha)
    return (y, alpha)


# speedup vs baseline: 1.6213x; 1.6213x over previous
import functools

import jax
import jax.numpy as jnp
from jax import lax
from jax.experimental import pallas as pl
from jax.experimental.pallas import tpu as pltpu
from jax.experimental.pallas import tpu_sc as plsc

NC = 2
NS = 16
NW = NC * NS
CHUNK = 128


def _sc_aggregate(x, srcp, dstp, zeros, cpw, n_pad):
    _, d = x.shape
    rps = n_pad // NS
    h = CHUNK // 2

    mesh = plsc.VectorSubcoreMesh(core_axis_name="c", subcore_axis_name="s")

    @functools.partial(
        pl.kernel,
        out_type=jax.ShapeDtypeStruct((NC, n_pad, d), jnp.float32),
        mesh=mesh,
        scratch_types=[
            pltpu.VMEM((cpw // 2, CHUNK), jnp.int32),
            pltpu.VMEM((cpw // 2, CHUNK), jnp.int32),
            pltpu.VMEM((2, CHUNK), jnp.int32),
            pltpu.VMEM((2, CHUNK), jnp.int32),
            pltpu.VMEM((CHUNK, d), jnp.float32),
            pltpu.VMEM((CHUNK, d), jnp.float32),
            pltpu.VMEM_SHARED((n_pad, d), jnp.float32),
            pltpu.SemaphoreType.DMA,
            pltpu.SemaphoreType.DMA,
        ],
    )
    def sc_agg(x_hbm, src_hbm, dst_hbm, z_hbm, out_hbm,
               src_v, dst_v, idx0, idx1, rows0, rows1, acc, sem0, sem1):
        c = lax.axis_index("c")
        s = lax.axis_index("s")
        wid = c * NS + s
        pltpu.sync_copy(z_hbm, acc.at[pl.ds(s * rps, rps)])
        pltpu.sync_copy(src_hbm.at[wid], src_v)
        pltpu.sync_copy(dst_hbm.at[wid], dst_v)
        plsc.subcore_barrier()

        def unpack(idx, row, off):
            for k in range(h // 16):
                ws = src_v[row, pl.ds(off + 16 * k, 16)]
                idx[0, pl.ds(16 * k, 16)] = ws & 0xFFFF
                idx[0, pl.ds(h + 16 * k, 16)] = lax.shift_right_logical(ws, 16)
                wd = dst_v[row, pl.ds(off + 16 * k, 16)]
                idx[1, pl.ds(16 * k, 16)] = wd & 0xFFFF
                idx[1, pl.ds(h + 16 * k, 16)] = lax.shift_right_logical(wd, 16)

        unpack(idx0, 0, 0)
        pltpu.async_copy(x_hbm.at[idx0.at[0]], rows0, sem0)
        unpack(idx1, 0, h)
        pltpu.async_copy(x_hbm.at[idx1.at[0]], rows1, sem1)

        def body(jj, carry):
            pltpu.make_async_copy(x_hbm.at[idx0.at[0]], rows0, sem0).wait()
            pltpu.sync_copy(rows0, acc.at[idx0.at[1]], add=True)
            unpack(idx0, jj + 1, 0)
            pltpu.async_copy(x_hbm.at[idx0.at[0]], rows0, sem0)
            pltpu.make_async_copy(x_hbm.at[idx1.at[0]], rows1, sem1).wait()
            pltpu.sync_copy(rows1, acc.at[idx1.at[1]], add=True)
            unpack(idx1, jj + 1, h)
            pltpu.async_copy(x_hbm.at[idx1.at[0]], rows1, sem1)
            return carry

        lax.fori_loop(0, cpw // 2 - 1, body, 0)
        pltpu.make_async_copy(x_hbm.at[idx0.at[0]], rows0, sem0).wait()
        pltpu.sync_copy(rows0, acc.at[idx0.at[1]], add=True)
        pltpu.make_async_copy(x_hbm.at[idx1.at[0]], rows1, sem1).wait()
        pltpu.sync_copy(rows1, acc.at[idx1.at[1]], add=True)

        plsc.subcore_barrier()
        pltpu.sync_copy(acc.at[pl.ds(s * rps, rps)],
                        out_hbm.at[c].at[pl.ds(s * rps, rps)])

    return sc_agg(x, srcp, dstp, zeros)


def _tc_mlp(x, parts, W1, b1, W2, b2, alpha):
    n, d = x.shape
    do = W2.shape[1]
    br = 1000

    def body(x_ref, p_ref, w1_ref, b1_ref, w2_ref, b2_ref, a_ref, o_ref):
        h = x_ref[...] + p_ref[0] + p_ref[1]
        h = jnp.dot(h, w1_ref[...], preferred_element_type=jnp.float32) + b1_ref[...]
        h = jnp.maximum(h, 0.0)
        y = jnp.dot(h, w2_ref[...], preferred_element_type=jnp.float32) + b2_ref[...]
        o_ref[...] = y * a_ref[0, 0]

    return pl.pallas_call(
        body,
        grid=(n // br,),
        in_specs=[
            pl.BlockSpec((br, d), lambda i: (i, 0)),
            pl.BlockSpec((NC, br, d), lambda i: (0, i, 0)),
            pl.BlockSpec((d, do), lambda i: (0, 0)),
            pl.BlockSpec((1, do), lambda i: (0, 0)),
            pl.BlockSpec((do, do), lambda i: (0, 0)),
            pl.BlockSpec((1, do), lambda i: (0, 0)),
            pl.BlockSpec((1, 1), lambda i: (0, 0)),
        ],
        out_specs=pl.BlockSpec((br, do), lambda i: (i, 0)),
        out_shape=jax.ShapeDtypeStruct((n, do), jnp.float32),
    )(x, parts, W1, b1.reshape(1, do), W2, b2.reshape(1, do), alpha.reshape(1, 1))


def kernel(x, edge_index, W1, b1, W2, b2, alpha):
    n, d = x.shape
    e = edge_index.shape[1]

    cpw = -(-e // (NW * CHUNK))
    cpw += cpw % 2
    e_pad = NW * cpw * CHUNK
    n_pad = -(-(n + 1) // (NS * 8)) * (NS * 8)

    src = edge_index[0].astype(jnp.int32)
    dst = edge_index[1].astype(jnp.int32)
    pad = e_pad - e
    src = jnp.concatenate([src, jnp.zeros((pad,), jnp.int32)])
    pad_dst = n + jnp.arange(pad, dtype=jnp.int32) % (n_pad - n)
    dst = jnp.concatenate([dst, pad_dst])

    def pack16(a):
        a3 = a.reshape(NW, cpw, CHUNK)
        p = a3[..., : CHUNK // 2] | (a3[..., CHUNK // 2:] << 16)
        return p.reshape(NW, cpw // 2, CHUNK)

    srcp = pack16(src)
    dstp = pack16(dst)
    zeros = jnp.zeros((n_pad // NS, d), jnp.float32)

    parts = _sc_aggregate(x, srcp, dstp, zeros, cpw, n_pad)
    y = _tc_mlp(x, parts, W1, b1, W2, b2, alpha)
    return (y, alpha)
